# baseline (speedup 1.0000x reference)
import functools

import numpy as np
import jax
import jax.numpy as jnp
from jax import lax
from jax.experimental import pallas as pl
from jax.experimental.pallas import tpu as pltpu
from jax.experimental.pallas import tpu_sc as plsc

_ROWS = 100000
_N_CHUNKS = 4
_CHUNK = 16
_DIM = _N_CHUNKS * _CHUNK

_HASH_A = [2654435761, 2246822519, 3266489917, 668265263]
_HASH_B = [374761393, 3144134277, 1013904223, 2773480762]

_NW = 32
_LANES = 16
_STAGE = 4096


def _build_sc_call(batch):
    rows_per_w = _DIM // _NW
    mesh = plsc.VectorSubcoreMesh(core_axis_name="c", subcore_axis_name="s")

    @functools.partial(
        pl.kernel,
        mesh=mesh,
        compiler_params=pltpu.CompilerParams(use_tc_tiling_on_sc=True,
                                             needs_layout_passes=False,
                                             disable_bounds_checks=True,
                                             disable_semaphore_checks=True),
        out_type=jax.ShapeDtypeStruct((_DIM, batch), jnp.float32),
        scratch_types=[
            pltpu.VMEM((_ROWS,), jnp.float32),
            pltpu.VMEM((batch,), jnp.int32),
            pltpu.VMEM((2, _STAGE), jnp.float32),
            pltpu.SemaphoreType.DMA,
            pltpu.SemaphoreType.DMA,
        ],
    )
    def sc_kernel(x_hbm, tbl_hbm, out_hbm, row_v, h_v, stage_v, sem, sem_out):
        w = lax.axis_index("s") * 2 + lax.axis_index("c")
        c = w // (_NW // _N_CHUNKS)
        k0 = (w % (_NW // _N_CHUNKS)) * rows_per_w

        cp0 = pltpu.async_copy(tbl_hbm.at[c, k0], row_v, sem)

        with jax.named_scope("xdma"):
            pltpu.sync_copy(x_hbm, h_v)

        def sel(vals):
            r = jnp.uint32(vals[3])
            for j in range(2, -1, -1):
                r = jnp.where(c == j, jnp.uint32(vals[j]), r)
            return r

        a_c = sel(_HASH_A)
        b_c = sel(_HASH_B)
        rows_mod = jnp.uint32(_ROWS)

        with jax.named_scope("hash"):
            @plsc.parallel_loop(0, batch // _LANES, unroll=8)
            def _hashv(i):
                xv = h_v[pl.ds(i * _LANES, _LANES)]
                h = plsc.bitcast(xv, jnp.uint32) * a_c + b_c
                h_v[pl.ds(i * _LANES, _LANES)] = plsc.bitcast(h % rows_mod,
                                                              jnp.int32)

        with jax.named_scope("row0wait"):
            cp0.wait()

        n_pieces = batch // _STAGE
        pending = [None, None]
        for r in range(rows_per_w):
            d = c * _CHUNK + k0 + r
            for piece in range(n_pieces):
                buf = (r * n_pieces + piece) % 2
                if pending[buf] is not None:
                    pending[buf].wait()

                with jax.named_scope(f"gat{r}_{piece}"):
                    @plsc.parallel_loop(0, _STAGE // _LANES, unroll=8)
                    def _gat(i):
                        idx = h_v[pl.ds(piece * _STAGE + i * _LANES, _LANES)]
                        stage_v[buf, pl.ds(i * _LANES, _LANES)] = (
                            plsc.load_gather(row_v, [idx]))

                pending[buf] = pltpu.async_copy(
                    stage_v.at[buf],
                    out_hbm.at[d, pl.ds(piece * _STAGE, _STAGE)], sem_out)
            if r + 1 < rows_per_w:
                with jax.named_scope("row1"):
                    pltpu.async_copy(tbl_hbm.at[c, k0 + r + 1], row_v,
                                     sem).wait()
        pending[0].wait()
        pending[1].wait()

    return sc_kernel


@jax.jit
def _run(x, table):
    batch = x.shape[0]
    tbl_t = jnp.transpose(table, (1, 2, 0))
    out_t = _build_sc_call(batch)(x, tbl_t)
    return out_t.T


def kernel(x, table):
    return _run(x, table)


# speedup vs baseline: 1.0865x; 1.0865x over previous
import functools

import numpy as np
import jax
import jax.numpy as jnp
from jax import lax
from jax.experimental import pallas as pl
from jax.experimental.pallas import tpu as pltpu
from jax.experimental.pallas import tpu_sc as plsc

_ROWS = 100000
_N_CHUNKS = 4
_CHUNK = 16
_DIM = _N_CHUNKS * _CHUNK

_HASH_A = [2654435761, 2246822519, 3266489917, 668265263]
_HASH_B = [374761393, 3144134277, 1013904223, 2773480762]

_NW = 32
_LANES = 16
_STAGE = 4096


def _build_sc_call(batch):
    rows_per_w = _DIM // _NW
    mesh = plsc.VectorSubcoreMesh(core_axis_name="c", subcore_axis_name="s")

    @functools.partial(
        pl.kernel,
        mesh=mesh,
        compiler_params=pltpu.CompilerParams(use_tc_tiling_on_sc=True,
                                             needs_layout_passes=False,
                                             disable_bounds_checks=True,
                                             disable_semaphore_checks=True),
        out_type=jax.ShapeDtypeStruct((_DIM, batch), jnp.float32),
        scratch_types=[
            pltpu.VMEM((_ROWS,), jnp.float32),
            pltpu.VMEM((batch,), jnp.int32),
            pltpu.VMEM((2, _STAGE), jnp.float32),
            pltpu.VMEM_SHARED((batch,), jnp.int32),
            pltpu.SemaphoreType.DMA,
            pltpu.SemaphoreType.DMA,
        ],
    )
    def sc_kernel(x_hbm, tbl_hbm, out_hbm, row_v, h_v, stage_v, x_sh,
                  sem, sem_out):
        w = lax.axis_index("s") * 2 + lax.axis_index("c")
        c = w // (_NW // _N_CHUNKS)
        k0 = (w % (_NW // _N_CHUNKS)) * rows_per_w

        cp0 = pltpu.async_copy(tbl_hbm.at[c, k0], row_v, sem)

        @pl.when(lax.axis_index("s") == 0)
        def _load_x():
            pltpu.sync_copy(x_hbm, x_sh)

        plsc.subcore_barrier()
        pltpu.sync_copy(x_sh, h_v)

        def sel(vals):
            r = jnp.uint32(vals[3])
            for j in range(2, -1, -1):
                r = jnp.where(c == j, jnp.uint32(vals[j]), r)
            return r

        a_c = sel(_HASH_A)
        b_c = sel(_HASH_B)
        rows_mod = jnp.uint32(_ROWS)

        @plsc.parallel_loop(0, batch // _LANES, unroll=8)
        def _hashv(i):
            xv = h_v[pl.ds(i * _LANES, _LANES)]
            h = plsc.bitcast(xv, jnp.uint32) * a_c + b_c
            h_v[pl.ds(i * _LANES, _LANES)] = plsc.bitcast(h % rows_mod,
                                                          jnp.int32)

        cp0.wait()

        n_pieces = batch // _STAGE
        pending = [None, None]
        for r in range(rows_per_w):
            d = c * _CHUNK + k0 + r
            for piece in range(n_pieces):
                buf = (r * n_pieces + piece) % 2
                if pending[buf] is not None:
                    pending[buf].wait()

                @plsc.parallel_loop(0, _STAGE // _LANES, unroll=8)
                def _gat(i):
                    idx = h_v[pl.ds(piece * _STAGE + i * _LANES, _LANES)]
                    stage_v[buf, pl.ds(i * _LANES, _LANES)] = (
                        plsc.load_gather(row_v, [idx]))

                pending[buf] = pltpu.async_copy(
                    stage_v.at[buf],
                    out_hbm.at[d, pl.ds(piece * _STAGE, _STAGE)], sem_out)
            if r + 1 < rows_per_w:
                pltpu.async_copy(tbl_hbm.at[c, k0 + r + 1], row_v, sem).wait()
        pending[0].wait()
        pending[1].wait()

    return sc_kernel


@jax.jit
def _run(x, table):
    batch = x.shape[0]
    tbl_t = jnp.transpose(table, (1, 2, 0))
    out_t = _build_sc_call(batch)(x, tbl_t)
    return out_t.T


def kernel(x, table):
    return _run(x, table)


# speedup vs baseline: 1.0866x; 1.0001x over previous
import functools

import jax
import jax.numpy as jnp
from jax import lax
from jax.experimental import pallas as pl
from jax.experimental.pallas import tpu as pltpu
from jax.experimental.pallas import tpu_sc as plsc

_ROWS = 100000
_N_CHUNKS = 4
_CHUNK = 16
_DIM = _N_CHUNKS * _CHUNK

_HASH_A = [2654435761, 2246822519, 3266489917, 668265263]
_HASH_B = [374761393, 3144134277, 1013904223, 2773480762]

_NW = 32
_LANES = 16
_STAGE = 4096


def _build_sc_call(batch):
    rows_per_w = _DIM // _NW
    mesh = plsc.VectorSubcoreMesh(core_axis_name="c", subcore_axis_name="s")

    @functools.partial(
        pl.kernel,
        mesh=mesh,
        compiler_params=pltpu.CompilerParams(use_tc_tiling_on_sc=True,
                                             needs_layout_passes=False,
                                             disable_bounds_checks=True,
                                             disable_semaphore_checks=True),
        out_type=jax.ShapeDtypeStruct((_DIM, batch), jnp.float32),
        scratch_types=[
            pltpu.VMEM((_ROWS,), jnp.float32),
            pltpu.VMEM((batch,), jnp.int32),
            pltpu.VMEM((2, _STAGE), jnp.float32),
            pltpu.VMEM_SHARED((batch,), jnp.int32),
            pltpu.SemaphoreType.DMA,
            pltpu.SemaphoreType.DMA,
        ],
    )
    def sc_kernel(x_hbm, tbl_hbm, out_hbm, row_v, h_v, stage_v, x_sh,
                  sem, sem_out):
        w = lax.axis_index("s") * 2 + lax.axis_index("c")
        c = w // (_NW // _N_CHUNKS)
        k0 = (w % (_NW // _N_CHUNKS)) * rows_per_w

        cp0 = pltpu.async_copy(tbl_hbm.at[c, k0], row_v, sem)

        @pl.when(lax.axis_index("s") == 0)
        def _load_x():
            pltpu.sync_copy(x_hbm, x_sh)

        plsc.subcore_barrier()
        pltpu.sync_copy(x_sh, h_v)

        def sel(vals):
            r = jnp.uint32(vals[3])
            for j in range(2, -1, -1):
                r = jnp.where(c == j, jnp.uint32(vals[j]), r)
            return r

        a_c = sel(_HASH_A)
        b_c = sel(_HASH_B)
        rows_mod = jnp.uint32(_ROWS)

        @plsc.parallel_loop(0, batch // _LANES, unroll=8)
        def _hashv(i):
            xv = h_v[pl.ds(i * _LANES, _LANES)]
            h = plsc.bitcast(xv, jnp.uint32) * a_c + b_c
            h_v[pl.ds(i * _LANES, _LANES)] = plsc.bitcast(h % rows_mod,
                                                          jnp.int32)

        cp0.wait()

        n_pieces = batch // _STAGE
        pending = [None, None]
        for r in range(rows_per_w):
            d = c * _CHUNK + k0 + r
            for piece in range(n_pieces):
                buf = (r * n_pieces + piece) % 2
                if pending[buf] is not None:
                    pending[buf].wait()

                @plsc.parallel_loop(0, _STAGE // _LANES, unroll=8)
                def _gat(i):
                    idx = h_v[pl.ds(piece * _STAGE + i * _LANES, _LANES)]
                    stage_v[buf, pl.ds(i * _LANES, _LANES)] = (
                        plsc.load_gather(row_v, [idx]))

                pending[buf] = pltpu.async_copy(
                    stage_v.at[buf],
                    out_hbm.at[d, pl.ds(piece * _STAGE, _STAGE)], sem_out)
            if r + 1 < rows_per_w:
                pltpu.async_copy(tbl_hbm.at[c, k0 + r + 1], row_v, sem).wait()
        pending[0].wait()
        pending[1].wait()

    return sc_kernel


@jax.jit
def _run(x, table):
    batch = x.shape[0]
    tbl_t = jnp.transpose(table, (1, 2, 0))
    out_t = _build_sc_call(batch)(x, tbl_t)
    return out_t.T


def kernel(x, table):
    return _run(x, table)
